# baseline (speedup 1.0000x reference)
import functools

import jax
import jax.numpy as jnp
from jax import lax
from jax.experimental import pallas as pl
from jax.experimental.pallas import tpu as pltpu
from jax.experimental.pallas import tpu_sc as plsc

N = 10000
E = 320000
D = 128
G = 128
NP = 10240
NC = 2
NS = 16
NW = NC * NS
EPT = E // NW
ROWS_PT = NP // NW
DH = D // 2
EPC = E // NS
CH = 128
NCH = 160
EPAD = NCH * CH

_sc_mesh = plsc.VectorSubcoreMesh(core_axis_name="c", subcore_axis_name="s")
_sc_params = pltpu.CompilerParams(needs_layout_passes=False)
_sc_params_nt = pltpu.CompilerParams(needs_layout_passes=False,
                                     use_tc_tiling_on_sc=False)


_W = NP // NS


@functools.partial(
    pl.kernel,
    out_type=jax.ShapeDtypeStruct((NC, NP), jnp.float32),
    mesh=_sc_mesh,
    compiler_params=_sc_params,
    scratch_types=[
        pltpu.VMEM_SHARED((NS, NP), jnp.float32),
        pltpu.VMEM((NP,), jnp.float32),
        pltpu.VMEM((EPT,), jnp.int32),
        pltpu.VMEM((NP,), jnp.float32),
    ],
)
def _deg_kernel(ecol_hbm, zeros_hbm, degp_out, shared, deg_v, ecol_v, strip_v):
    c = lax.axis_index("c")
    s = lax.axis_index("s")
    wid = c * NS + s
    pltpu.sync_copy(zeros_hbm.at[pl.ds(0, NP)], deg_v)
    pltpu.sync_copy(ecol_hbm.at[pl.ds(wid * EPT, EPT)], ecol_v)
    ones = jnp.full((16,), 1.0, dtype=jnp.float32)

    def body(i, carry):
        idx = ecol_v[pl.ds(i * 16, 16)]
        plsc.addupdate_scatter(deg_v, [idx], ones)
        return carry

    lax.fori_loop(0, EPT // 16, body, 0)
    pltpu.sync_copy(deg_v, shared.at[s])
    plsc.subcore_barrier()
    for p in range(NS):
        pltpu.sync_copy(shared.at[p, pl.ds(s * _W, _W)],
                        strip_v.at[pl.ds(p * _W, _W)])

    def sum_body(v, carry):
        acc = strip_v[pl.ds(v * 16, 16)]
        for p in range(1, NS):
            acc = acc + strip_v[pl.ds(p * _W + v * 16, 16)]
        deg_v[pl.ds(s * _W + v * 16, 16)] = acc
        return carry

    lax.fori_loop(0, _W // 16, sum_body, 0)
    pltpu.sync_copy(deg_v.at[pl.ds(s * _W, _W)],
                    degp_out.at[c, pl.ds(s * _W, _W)])


def _xw_body(x_ref, w1_ref, degp_ref, y_ref, dinv_ref):
    deg = jnp.sum(degp_ref[...], axis=0) + 1.0
    dinv = lax.rsqrt(deg)
    xw = jnp.dot(x_ref[...], w1_ref[...], preferred_element_type=jnp.float32)
    yv = xw * dinv
    y_ref[0] = yv[:, :DH]
    y_ref[1] = yv[:, DH:]
    dinv_ref[...] = dinv


_BB = 2000
_xw_call = pl.pallas_call(
    _xw_body,
    grid=(N // _BB,),
    in_specs=[
        pl.BlockSpec((_BB, D), lambda i: (i, 0)),
        pl.BlockSpec((D, D), lambda i: (0, 0)),
        pl.BlockSpec((NC, _BB, 1), lambda i: (0, i, 0)),
    ],
    out_specs=[
        pl.BlockSpec((NC, _BB, DH), lambda i: (0, i, 0)),
        pl.BlockSpec((_BB, 1), lambda i: (i, 0)),
    ],
    out_shape=[
        jax.ShapeDtypeStruct((NC, N, DH), jnp.float32),
        jax.ShapeDtypeStruct((N, 1), jnp.float32),
    ],
)


@functools.partial(
    pl.kernel,
    out_type=jax.ShapeDtypeStruct((NC, NP, DH), jnp.float32),
    mesh=_sc_mesh,
    compiler_params=_sc_params_nt,
    scratch_types=[
        pltpu.VMEM_SHARED((NP, DH), jnp.float32),
        pltpu.VMEM((EPAD,), jnp.int32),
        pltpu.VMEM((NCH, CH), jnp.int32),
        pltpu.VMEM((CH, DH), jnp.float32),
        pltpu.VMEM((CH, DH), jnp.float32),
        pltpu.VMEM((CH, DH), jnp.float32),
        pltpu.VMEM((CH, DH), jnp.float32),
        pltpu.SemaphoreType.DMA,
        pltpu.SemaphoreType.DMA,
        pltpu.SemaphoreType.DMA,
        pltpu.SemaphoreType.DMA,
        pltpu.SemaphoreType.DMA,
        pltpu.SemaphoreType.DMA,
        pltpu.SemaphoreType.DMA,
        pltpu.SemaphoreType.DMA,
    ],
)
def _scatter_kernel(ys_hbm, erow_hbm, ecf_hbm, zeros_hbm, s_out,
                    shared, ir_all, ic_all, rb0, rb1, rb2, rb3,
                    g0, g1, g2, g3, s0, s1, s2, s3):
    c = lax.axis_index("c")
    s = lax.axis_index("s")
    rbs = (rb0, rb1, rb2, rb3)
    gsems = (g0, g1, g2, g3)
    ssems = (s0, s1, s2, s3)
    pltpu.sync_copy(zeros_hbm, rb0)
    for q in range(5):
        pltpu.sync_copy(rb0, shared.at[pl.ds(s * 640 + q * CH, CH)])
    pltpu.sync_copy(erow_hbm.at[pl.ds(s * EPC, EPC)], ir_all.at[pl.ds(0, EPC)])
    zero16 = jnp.zeros((16,), jnp.int32)
    for m in range((EPAD - EPC) // 16):
        ir_all[pl.ds(EPC + m * 16, 16)] = zero16

    def bias_body(v, carry):
        sl = pl.ds(v * 16, 16)
        ir_all[sl] = ir_all[sl] + c * N
        return carry

    lax.fori_loop(0, EPAD // 16, bias_body, 0)
    pltpu.sync_copy(ecf_hbm.at[s], ic_all)
    plsc.subcore_barrier()

    def gather(i, q):
        return pltpu.async_copy(ys_hbm.at[ir_all.at[pl.ds(i * CH, CH)]],
                                rbs[q], gsems[q])

    def scatter(i, q):
        return pltpu.async_copy(rbs[q], shared.at[ic_all.at[i]], ssems[q],
                                add=True)

    def wait_gather(q):
        pltpu.make_async_copy(ys_hbm.at[ir_all.at[pl.ds(0, CH)]],
                              rbs[q], gsems[q]).wait()

    def wait_scatter(q):
        pltpu.make_async_copy(rbs[q], shared.at[ic_all.at[0]], ssems[q]).wait()

    for q in range(4):
        gather(q, q)

    def body(k, carry):
        i0 = 4 * k
        for q in range(4):
            wait_gather(q)
            scatter(i0 + q, q)
        for q in range(4):
            wait_scatter(q)
            gather(i0 + 4 + q, q)
        return carry

    lax.fori_loop(0, NCH // 4 - 1, body, 0)
    for q in range(4):
        wait_gather(q)
        scatter(NCH - 4 + q, q)
    for q in range(4):
        wait_scatter(q)
    plsc.subcore_barrier()

    pltpu.sync_copy(shared.at[pl.ds(s * 640, 640)],
                    s_out.at[c, pl.ds(s * 640, 640)])


_RCH = 80


@functools.partial(
    pl.kernel,
    out_type=jax.ShapeDtypeStruct((NW, G * D), jnp.float32),
    mesh=_sc_mesh,
    compiler_params=_sc_params,
    scratch_types=[
        pltpu.VMEM((G * D,), jnp.float32),
        pltpu.VMEM((_RCH * DH,), jnp.float32),
        pltpu.VMEM((_RCH * DH,), jnp.float32),
        pltpu.VMEM((_RCH * DH,), jnp.float32),
        pltpu.VMEM((_RCH * DH,), jnp.float32),
        pltpu.VMEM((ROWS_PT,), jnp.float32),
        pltpu.VMEM((ROWS_PT,), jnp.int32),
        pltpu.VMEM((D,), jnp.float32),
    ],
)
def _segmax_kernel(s_hbm, y_hbm, dinv_hbm, batch_hbm, b1_hbm, zeros_hbm, hp_out,
                   hp, s0b, s1b, y0b, y1b, dvb, btb, b1b):
    c = lax.axis_index("c")
    s = lax.axis_index("s")
    wid = c * NS + s
    tbase = wid * ROWS_PT
    pltpu.sync_copy(zeros_hbm.at[pl.ds(0, G * D)], hp)
    pltpu.sync_copy(b1_hbm, b1b)
    pltpu.sync_copy(dinv_hbm.at[pl.ds(tbase, ROWS_PT)], dvb)
    pltpu.sync_copy(batch_hbm.at[pl.ds(tbase, ROWS_PT)], btb)
    nch = jnp.clip((N - tbase) // _RCH, 0, ROWS_PT // _RCH)

    def chunk_body(ch, carry):
        base = tbase + ch * _RCH
        pltpu.sync_copy(s_hbm.at[0, pl.ds(base * DH, _RCH * DH)], s0b)
        pltpu.sync_copy(s_hbm.at[1, pl.ds(base * DH, _RCH * DH)], s1b)
        pltpu.sync_copy(y_hbm.at[0, pl.ds(base * DH, _RCH * DH)], y0b)
        pltpu.sync_copy(y_hbm.at[1, pl.ds(base * DH, _RCH * DH)], y1b)

        def grp_body(gi, carry2):
            bt16 = btb[pl.ds(ch * _RCH + gi * 16, 16)]
            dv16 = dvb[pl.ds(ch * _RCH + gi * 16, 16)]
            gofs = bt16 * D
            rbase = gi * 16 * DH
            for k in range(16):
                g = gofs[k]
                dv = dv16[k]
                for j in range(8):
                    half, jj = (0, j) if j < 4 else (1, j - 4)
                    sb = (s0b, s1b)[half]
                    yb = (y0b, y1b)[half]
                    rsl = pl.ds(rbase + k * DH + jj * 16, 16)
                    hv = (sb[rsl] + yb[rsl]) * dv + b1b[pl.ds(j * 16, 16)]
                    hv = jnp.maximum(hv, 0.0)
                    hsl = pl.ds(g + j * 16, 16)
                    hp[hsl] = jnp.maximum(hp[hsl], hv)
            return carry2

        lax.fori_loop(0, _RCH // 16, grp_body, 0)
        return carry

    lax.fori_loop(0, nch, chunk_body, 0)
    pltpu.sync_copy(hp, hp_out.at[wid])


def _final_body(hp_ref, w2_ref, b2_ref, out_ref):
    hp = jnp.max(hp_ref[...], axis=0)
    o = jnp.dot(hp, w2_ref[...], preferred_element_type=jnp.float32) + b2_ref[...]
    m = jnp.max(o, axis=-1, keepdims=True)
    z = o - m
    out_ref[...] = z - jnp.log(jnp.sum(jnp.exp(z), axis=-1, keepdims=True))


_final_call = pl.pallas_call(
    _final_body,
    out_shape=jax.ShapeDtypeStruct((G, 2), jnp.float32),
)


def kernel(x, edge_index, batch, W1, b1, W2, b2):
    batch_pad = jnp.pad(batch, (0, NP - N))
    zeros_ch = jnp.zeros((CH, DH), jnp.float32)
    zeros_flat = jnp.zeros((G * D,), jnp.float32)
    ecf = jnp.pad(edge_index[1].reshape(NS, EPC), ((0, 0), (0, EPAD - EPC)),
                  constant_values=NP - 1).reshape(NS, NCH, CH)
    degp = _deg_kernel(edge_index[1], zeros_flat)
    ys, dinv = _xw_call(x, W1, degp.reshape(NC, NP, 1))
    s_part = _scatter_kernel(ys.reshape(NC * N, DH), edge_index[0], ecf, zeros_ch)
    dinv_pad = jnp.pad(dinv.reshape(N), (0, NP - N))
    hp_parts = _segmax_kernel(s_part.reshape(NC, NP * DH), ys.reshape(NC, N * DH),
                              dinv_pad, batch_pad, b1, zeros_flat)
    return _final_call(hp_parts.reshape(NW, G, D), W2, b2.reshape(1, 2))
